# baseline (speedup 1.0000x reference)
import functools
import jax
import jax.numpy as jnp
from jax import lax
from jax.experimental import pallas as pl
from jax.experimental.pallas import tpu as pltpu
from jax.experimental.pallas import tpu_sc as plsc

_B, _C, _WH = 4, 192, 56
_XW = _WH + 1
_HEADS, _HD, _NS = 8, 24, 9
_N = _WH * _WH
_BH = _B * _HEADS
_P = 112
_NCH = _N // _P
_SCALE = _HD ** -0.5
_MB = 1568


def _mm_kernel(a_ref, b_ref, o_ref):
    o_ref[...] = jnp.dot(a_ref[...], b_ref[...], preferred_element_type=jnp.float32)


def _mm(a, b):
    m, k = a.shape
    _, n = b.shape
    return pl.pallas_call(
        _mm_kernel,
        grid=(m // _MB,),
        in_specs=[
            pl.BlockSpec((_MB, k), lambda i: (i, 0)),
            pl.BlockSpec((k, n), lambda i: (0, 0)),
        ],
        out_specs=pl.BlockSpec((_MB, n), lambda i: (i, 0)),
        out_shape=jax.ShapeDtypeStruct((m, n), jnp.float32),
    )(a, b)


def _mm_bias_kernel(a_ref, b_ref, bias_ref, o_ref):
    o_ref[...] = (
        jnp.dot(a_ref[...], b_ref[...], preferred_element_type=jnp.float32)
        + bias_ref[...]
    )


def _mm_bias(a, b, bias):
    m, k = a.shape
    _, n = b.shape
    return pl.pallas_call(
        _mm_bias_kernel,
        grid=(m // _MB,),
        in_specs=[
            pl.BlockSpec((_MB, k), lambda i: (i, 0)),
            pl.BlockSpec((k, n), lambda i: (0, 0)),
            pl.BlockSpec((1, n), lambda i: (0, 0)),
        ],
        out_specs=pl.BlockSpec((_MB, n), lambda i: (i, 0)),
        out_shape=jax.ShapeDtypeStruct((m, n), jnp.float32),
    )(a, b, bias)


def _idxw_kernel(oy_ref, ox_ref, idx_ref, w_ref):
    jc = pl.program_id(1)
    oy = oy_ref[0, 0]
    ox = ox_ref[0, 0]
    s = lax.broadcasted_iota(jnp.int32, (_NS, _P), 0)
    n = jc * _P + lax.broadcasted_iota(jnp.int32, (_NS, _P), 1)
    rowf = (n // _WH).astype(jnp.float32)
    colf = (n % _WH).astype(jnp.float32)
    ky = (s // 3 - 1).astype(jnp.float32)
    kx = (s % 3 - 1).astype(jnp.float32)
    py = rowf + ky + oy
    px = colf + kx + ox
    y0 = jnp.floor(py)
    x0 = jnp.floor(px)
    wy1 = py - y0
    wy0 = 1.0 - wy1
    wx1 = px - x0
    wx0 = 1.0 - wx1
    lim = float(_WH - 1)
    y1 = y0 + 1.0
    x1 = x0 + 1.0
    vy0 = ((y0 >= 0) & (y0 <= lim)).astype(jnp.float32)
    vy1 = ((y1 >= 0) & (y1 <= lim)).astype(jnp.float32)
    vx0 = ((x0 >= 0) & (x0 <= lim)).astype(jnp.float32)
    vx1 = ((x1 >= 0) & (x1 <= lim)).astype(jnp.float32)
    xi = jnp.clip(x1, 0.0, float(_WH)).astype(jnp.int32)
    r0 = jnp.clip(y0, 0.0, lim).astype(jnp.int32)
    r1 = jnp.clip(y1, 0.0, lim).astype(jnp.int32)
    idx_ref[0, 0] = jnp.concatenate([r0 * _XW + xi, r1 * _XW + xi], axis=0)
    w_ref[0, 0] = jnp.concatenate(
        [wy0 * vy0, wy1 * vy1, wx0 * vx0, wx1 * vx1], axis=0
    )


def _idxw(oy, ox):
    return pl.pallas_call(
        _idxw_kernel,
        grid=(_BH, _NCH),
        in_specs=[
            pl.BlockSpec((1, 1, _NS, _P), lambda b, j: (b, j, 0, 0)),
            pl.BlockSpec((1, 1, _NS, _P), lambda b, j: (b, j, 0, 0)),
        ],
        out_specs=[
            pl.BlockSpec((1, 1, 2 * _NS, _P), lambda b, j: (b, j, 0, 0)),
            pl.BlockSpec((1, 1, 4 * _NS, _P), lambda b, j: (b, j, 0, 0)),
        ],
        out_shape=[
            jax.ShapeDtypeStruct((_BH, _NCH, 2 * _NS, _P), jnp.int32),
            jax.ShapeDtypeStruct((_BH, _NCH, 4 * _NS, _P), jnp.float32),
        ],
    )(oy, ox)


_SC_MESH = plsc.VectorSubcoreMesh(core_axis_name="c", subcore_axis_name="s")
_SC_PARAMS = pltpu.CompilerParams(needs_layout_passes=False)


def _bilin(tab, dsplat, iv, wx0, wx1):
    g = plsc.load_gather(tab, [dsplat, iv])
    lo, hi = plsc.unpack(
        plsc.bitcast(g, jnp.bfloat16), format=plsc.PackFormat.INTERLEAVED
    )
    return wx0 * lo + wx1 * hi


@functools.partial(
    pl.kernel,
    out_type=jax.ShapeDtypeStruct((_BH, _NCH, _HD, _P), jnp.float32),
    mesh=_SC_MESH,
    compiler_params=_SC_PARAMS,
    scratch_types=[
        pltpu.VMEM((_HD, _WH * _XW), jnp.float32),
        pltpu.VMEM((_NS - 1, _N), jnp.float32),
        pltpu.VMEM((_HD + 4 * _NS, _P), jnp.float32),
        pltpu.VMEM((2 * _NS, _P), jnp.int32),
    ],
)
def _sc_attn(ktab_h, vtab_h, q_h, idx_h, w_h, o_h, tab, pball, fb, ib):
    wid = lax.axis_index("s") * 2 + lax.axis_index("c")
    pltpu.sync_copy(ktab_h.at[wid], tab)

    def chunk_a(jc, carry):
        pltpu.sync_copy(q_h.at[wid, jc], fb.at[pl.ds(0, _HD)])
        pltpu.sync_copy(idx_h.at[wid, jc], ib)
        pltpu.sync_copy(w_h.at[wid, jc], fb.at[pl.ds(_HD, 4 * _NS)])

        def tile(t, carry2):
            sl = pl.ds(t * 16, 16)
            qv = [fb[d, sl] for d in range(_HD)]
            logits = []
            for s in range(_NS):
                iv0 = ib[s, sl]
                iv1 = ib[_NS + s, sl]
                wy0 = fb[_HD + s, sl]
                wy1 = fb[_HD + _NS + s, sl]
                wx0 = fb[_HD + 2 * _NS + s, sl]
                wx1 = fb[_HD + 3 * _NS + s, sl]
                rd0 = None
                rd1 = None
                for d in range(_HD):
                    dsp = jnp.full((16,), d, jnp.int32)
                    v0 = _bilin(tab, dsp, iv0, wx0, wx1)
                    t0 = qv[d] * v0
                    rd0 = t0 if rd0 is None else rd0 + t0
                    v1 = _bilin(tab, dsp, iv1, wx0, wx1)
                    t1 = qv[d] * v1
                    rd1 = t1 if rd1 is None else rd1 + t1
                logits.append((wy0 * rd0 + wy1 * rd1) * _SCALE)
            m = logits[0]
            for s in range(1, _NS):
                m = jnp.maximum(m, logits[s])
            es = [jnp.exp(l - m) for l in logits]
            tot = es[0]
            for s in range(1, _NS):
                tot = tot + es[s]
            gsl = pl.ds(jc * _P + t * 16, 16)
            for s in range(_NS - 1):
                pball[s, gsl] = es[s] / tot
            return carry2

        lax.fori_loop(0, _P // 16, tile, 0)
        return carry

    lax.fori_loop(0, _NCH, chunk_a, 0)
    pltpu.sync_copy(vtab_h.at[wid], tab)

    def chunk_b(jc, carry):
        pltpu.sync_copy(idx_h.at[wid, jc], ib)
        pltpu.sync_copy(w_h.at[wid, jc], fb.at[pl.ds(_HD, 4 * _NS)])

        def tile(t, carry2):
            sl = pl.ds(t * 16, 16)
            gsl = pl.ds(jc * _P + t * 16, 16)
            pv = [pball[s, gsl] for s in range(_NS - 1)]
            plast = 1.0 - pv[0]
            for s in range(1, _NS - 1):
                plast = plast - pv[s]
            pv.append(plast)
            outs = [None] * _HD
            for s in range(_NS):
                iv0 = ib[s, sl]
                iv1 = ib[_NS + s, sl]
                wy0 = fb[_HD + s, sl]
                wy1 = fb[_HD + _NS + s, sl]
                wx0 = fb[_HD + 2 * _NS + s, sl]
                wx1 = fb[_HD + 3 * _NS + s, sl]
                c0 = pv[s] * wy0
                c1 = pv[s] * wy1
                for d in range(_HD):
                    dsp = jnp.full((16,), d, jnp.int32)
                    t0 = c0 * _bilin(tab, dsp, iv0, wx0, wx1)
                    outs[d] = t0 if outs[d] is None else outs[d] + t0
                    outs[d] = outs[d] + c1 * _bilin(tab, dsp, iv1, wx0, wx1)
            for d in range(_HD):
                fb[d, sl] = outs[d]
            return carry2

        lax.fori_loop(0, _P // 16, tile, 0)
        pltpu.sync_copy(fb.at[pl.ds(0, _HD)], o_h.at[wid, jc])
        return carry

    lax.fori_loop(0, _NCH, chunk_b, 0)


def _pack_pairs(t4):
    lo = jnp.pad(t4, ((0, 0), (0, 0), (0, 0), (1, 0)))
    hi = jnp.pad(t4, ((0, 0), (0, 0), (0, 0), (0, 1)))
    lo16 = lax.bitcast_convert_type(lo.astype(jnp.bfloat16), jnp.uint16)
    hi16 = lax.bitcast_convert_type(hi.astype(jnp.bfloat16), jnp.uint16)
    word = lo16.astype(jnp.uint32) | (hi16.astype(jnp.uint32) << 16)
    return lax.bitcast_convert_type(word, jnp.float32).reshape(
        _BH, _HD, _WH * _XW
    )


def kernel(x, qkv_w, offset_w, proj_w, proj_b):
    B, C, W, H = x.shape
    heads, hd, ns = _HEADS, _HD, _NS
    N = W * H

    eye = jnp.eye(heads, dtype=jnp.float32)
    blk = eye[:, None, :, None] * offset_w.T[None, :, None, :]
    blk = blk.reshape(C, heads * 2 * ns)
    wcat = jnp.concatenate([qkv_w.T, blk], axis=1)

    xf = jnp.transpose(x, (0, 2, 3, 1)).reshape(B * N, C)
    y = _mm(xf, wcat)

    q = y[:, 0:C].reshape(B, N, heads, hd)
    k = y[:, C:2 * C].reshape(B, N, heads, hd)
    v = y[:, 2 * C:3 * C].reshape(B, N, heads, hd)
    off = y[:, 3 * C:].reshape(B, N, heads, 2 * ns)

    ktab = _pack_pairs(jnp.transpose(k, (0, 2, 3, 1)).reshape(_BH, hd, W, H))
    vtab = _pack_pairs(jnp.transpose(v, (0, 2, 3, 1)).reshape(_BH, hd, W, H))
    q_cm = jnp.transpose(
        q.reshape(B, _NCH, _P, heads, hd), (0, 3, 1, 4, 2)
    ).reshape(_BH, _NCH, hd, _P)
    oy = jnp.transpose(off[..., 0::2], (0, 2, 3, 1)).reshape(_BH, ns, _NCH, _P)
    ox = jnp.transpose(off[..., 1::2], (0, 2, 3, 1)).reshape(_BH, ns, _NCH, _P)
    oy = jnp.transpose(oy, (0, 2, 1, 3))
    ox = jnp.transpose(ox, (0, 2, 1, 3))

    idx_cm, w_cm = _idxw(oy, ox)
    out_cm = _sc_attn(ktab, vtab, q_cm, idx_cm, w_cm)

    out_f = jnp.transpose(
        out_cm.reshape(B, heads, _NCH, hd, _P), (0, 2, 4, 1, 3)
    ).reshape(B * N, C)
    fin = _mm_bias(out_f, proj_w.T, proj_b.reshape(1, C))
    fin = fin.reshape(B, W, H, C)
    return jnp.transpose(fin, (0, 3, 1, 2))


# speedup vs baseline: 1.5428x; 1.5428x over previous
import functools
import jax
import jax.numpy as jnp
from jax import lax
from jax.experimental import pallas as pl
from jax.experimental.pallas import tpu as pltpu
from jax.experimental.pallas import tpu_sc as plsc

_B, _C, _WH = 4, 192, 56
_XW = _WH + 1
_HEADS, _HD, _NS = 8, 24, 9
_N = _WH * _WH
_BH = _B * _HEADS
_P = 112
_NCH = _N // _P
_SCALE = _HD ** -0.5
_MB = 1568


def _mm_kernel(a_ref, b_ref, o_ref):
    o_ref[...] = jnp.dot(a_ref[...], b_ref[...], preferred_element_type=jnp.float32)


def _mm(a, b):
    m, k = a.shape
    _, n = b.shape
    return pl.pallas_call(
        _mm_kernel,
        grid=(m // _MB,),
        in_specs=[
            pl.BlockSpec((_MB, k), lambda i: (i, 0)),
            pl.BlockSpec((k, n), lambda i: (0, 0)),
        ],
        out_specs=pl.BlockSpec((_MB, n), lambda i: (i, 0)),
        out_shape=jax.ShapeDtypeStruct((m, n), jnp.float32),
    )(a, b)


def _mm_bias_kernel(a_ref, b_ref, bias_ref, o_ref):
    o_ref[...] = (
        jnp.dot(a_ref[...], b_ref[...], preferred_element_type=jnp.float32)
        + bias_ref[...]
    )


def _mm_bias(a, b, bias):
    m, k = a.shape
    _, n = b.shape
    return pl.pallas_call(
        _mm_bias_kernel,
        grid=(m // _MB,),
        in_specs=[
            pl.BlockSpec((_MB, k), lambda i: (i, 0)),
            pl.BlockSpec((k, n), lambda i: (0, 0)),
            pl.BlockSpec((1, n), lambda i: (0, 0)),
        ],
        out_specs=pl.BlockSpec((_MB, n), lambda i: (i, 0)),
        out_shape=jax.ShapeDtypeStruct((m, n), jnp.float32),
    )(a, b, bias)


def _idxw_kernel(oy_ref, ox_ref, idx_ref, w_ref):
    oy = oy_ref[0]
    ox = ox_ref[0]
    shape = (_NCH, _NS, _P)
    jc = lax.broadcasted_iota(jnp.int32, shape, 0).astype(jnp.float32)
    sf = lax.broadcasted_iota(jnp.int32, shape, 1).astype(jnp.float32)
    i = lax.broadcasted_iota(jnp.int32, shape, 2).astype(jnp.float32)
    half = (i >= float(_WH)).astype(jnp.float32)
    rowf = 2.0 * jc + half
    colf = i - float(_WH) * half
    sdiv3 = jnp.floor(sf * (1.0 / 3.0))
    ky = sdiv3 - 1.0
    kx = sf - 3.0 * sdiv3 - 1.0
    py = rowf + ky + oy
    px = colf + kx + ox
    y0 = jnp.floor(py)
    x0 = jnp.floor(px)
    wy1 = py - y0
    wy0 = 1.0 - wy1
    wx1 = px - x0
    wx0 = 1.0 - wx1
    lim = float(_WH - 1)
    y1 = y0 + 1.0
    x1 = x0 + 1.0
    vy0 = ((y0 >= 0) & (y0 <= lim)).astype(jnp.float32)
    vy1 = ((y1 >= 0) & (y1 <= lim)).astype(jnp.float32)
    vx0 = ((x0 >= 0) & (x0 <= lim)).astype(jnp.float32)
    vx1 = ((x1 >= 0) & (x1 <= lim)).astype(jnp.float32)
    xi = jnp.clip(x1, 0.0, float(_WH)).astype(jnp.int32)
    r0 = jnp.clip(y0, 0.0, lim).astype(jnp.int32)
    r1 = jnp.clip(y1, 0.0, lim).astype(jnp.int32)
    idx_ref[0] = jnp.concatenate([r0 * _XW + xi, r1 * _XW + xi], axis=1)
    w_ref[0] = jnp.concatenate(
        [wy0 * vy0, wy1 * vy1, wx0 * vx0, wx1 * vx1], axis=1
    )


def _idxw(oy, ox):
    return pl.pallas_call(
        _idxw_kernel,
        grid=(_BH,),
        in_specs=[
            pl.BlockSpec((1, _NCH, _NS, _P), lambda b: (b, 0, 0, 0)),
            pl.BlockSpec((1, _NCH, _NS, _P), lambda b: (b, 0, 0, 0)),
        ],
        out_specs=[
            pl.BlockSpec((1, _NCH, 2 * _NS, _P), lambda b: (b, 0, 0, 0)),
            pl.BlockSpec((1, _NCH, 4 * _NS, _P), lambda b: (b, 0, 0, 0)),
        ],
        out_shape=[
            jax.ShapeDtypeStruct((_BH, _NCH, 2 * _NS, _P), jnp.int32),
            jax.ShapeDtypeStruct((_BH, _NCH, 4 * _NS, _P), jnp.float32),
        ],
    )(oy, ox)


_SC_MESH = plsc.VectorSubcoreMesh(core_axis_name="c", subcore_axis_name="s")
_SC_PARAMS = pltpu.CompilerParams(needs_layout_passes=False)


def _pair(tab, dsplat, iv):
    g = plsc.load_gather(tab, [dsplat, iv])
    return plsc.unpack(
        plsc.bitcast(g, jnp.bfloat16), format=plsc.PackFormat.INTERLEAVED
    )


@functools.partial(
    pl.kernel,
    out_type=jax.ShapeDtypeStruct((_BH, _NCH, _HD, _P), jnp.float32),
    mesh=_SC_MESH,
    compiler_params=_SC_PARAMS,
    scratch_types=[
        pltpu.VMEM((_HD, _WH * _XW), jnp.float32),
        pltpu.VMEM((_NS - 1, _N), jnp.float32),
        pltpu.VMEM((_HD + 4 * _NS, _P), jnp.float32),
        pltpu.VMEM((2 * _NS, _P), jnp.int32),
    ],
)
def _sc_attn(ktab_h, vtab_h, q_h, idx_h, w_h, o_h, tab, pball, fb, ib):
    wid = lax.axis_index("s") * 2 + lax.axis_index("c")
    pltpu.sync_copy(ktab_h.at[wid], tab)

    def chunk_a(jc, carry):
        pltpu.sync_copy(q_h.at[wid, jc], fb.at[pl.ds(0, _HD)])
        pltpu.sync_copy(idx_h.at[wid, jc], ib)
        pltpu.sync_copy(w_h.at[wid, jc], fb.at[pl.ds(_HD, 4 * _NS)])

        def tile(t, carry2):
            sl = pl.ds(t * 16, 16)
            qv = [fb[d, sl] for d in range(_HD)]
            logits = []
            for s in range(_NS):
                iv0 = ib[s, sl]
                iv1 = ib[_NS + s, sl]
                wy0 = fb[_HD + s, sl]
                wy1 = fb[_HD + _NS + s, sl]
                wx0 = fb[_HD + 2 * _NS + s, sl]
                wx1 = fb[_HD + 3 * _NS + s, sl]
                acc = [None] * 4
                for d in range(_HD):
                    dsp = jnp.full((16,), d, jnp.int32)
                    lo0, hi0 = _pair(tab, dsp, iv0)
                    lo1, hi1 = _pair(tab, dsp, iv1)
                    for j, g in enumerate((lo0, hi0, lo1, hi1)):
                        t0 = qv[d] * g
                        acc[j] = t0 if acc[j] is None else acc[j] + t0
                row0 = wx0 * acc[0] + wx1 * acc[1]
                row1 = wx0 * acc[2] + wx1 * acc[3]
                logits.append((wy0 * row0 + wy1 * row1) * _SCALE)
            m = logits[0]
            for s in range(1, _NS):
                m = jnp.maximum(m, logits[s])
            es = [jnp.exp(l - m) for l in logits]
            tot = es[0]
            for s in range(1, _NS):
                tot = tot + es[s]
            gsl = pl.ds(jc * _P + t * 16, 16)
            for s in range(_NS - 1):
                pball[s, gsl] = es[s] / tot
            return carry2

        lax.fori_loop(0, _P // 16, tile, 0)
        return carry

    lax.fori_loop(0, _NCH, chunk_a, 0)
    pltpu.sync_copy(vtab_h.at[wid], tab)

    def chunk_b(jc, carry):
        pltpu.sync_copy(idx_h.at[wid, jc], ib)
        pltpu.sync_copy(w_h.at[wid, jc], fb.at[pl.ds(_HD, 4 * _NS)])

        def tile(t, carry2):
            sl = pl.ds(t * 16, 16)
            gsl = pl.ds(jc * _P + t * 16, 16)
            pv = [pball[s, gsl] for s in range(_NS - 1)]
            plast = 1.0 - pv[0]
            for s in range(1, _NS - 1):
                plast = plast - pv[s]
            pv.append(plast)
            outs = [None] * _HD
            for s in range(_NS):
                iv0 = ib[s, sl]
                iv1 = ib[_NS + s, sl]
                wy0 = fb[_HD + s, sl]
                wy1 = fb[_HD + _NS + s, sl]
                wx0 = fb[_HD + 2 * _NS + s, sl]
                wx1 = fb[_HD + 3 * _NS + s, sl]
                c0 = pv[s] * wy0
                c1 = pv[s] * wy1
                cw = (c0 * wx0, c0 * wx1, c1 * wx0, c1 * wx1)
                for d in range(_HD):
                    dsp = jnp.full((16,), d, jnp.int32)
                    lo0, hi0 = _pair(tab, dsp, iv0)
                    lo1, hi1 = _pair(tab, dsp, iv1)
                    t0 = cw[0] * lo0
                    outs[d] = t0 if outs[d] is None else outs[d] + t0
                    outs[d] = outs[d] + cw[1] * hi0
                    outs[d] = outs[d] + cw[2] * lo1
                    outs[d] = outs[d] + cw[3] * hi1
            for d in range(_HD):
                fb[d, sl] = outs[d]
            return carry2

        lax.fori_loop(0, _P // 16, tile, 0)
        pltpu.sync_copy(fb.at[pl.ds(0, _HD)], o_h.at[wid, jc])
        return carry

    lax.fori_loop(0, _NCH, chunk_b, 0)


def _pack_pairs(t4):
    lo = jnp.pad(t4, ((0, 0), (0, 0), (0, 0), (1, 0)))
    hi = jnp.pad(t4, ((0, 0), (0, 0), (0, 0), (0, 1)))
    lo16 = lax.bitcast_convert_type(lo.astype(jnp.bfloat16), jnp.uint16)
    hi16 = lax.bitcast_convert_type(hi.astype(jnp.bfloat16), jnp.uint16)
    word = lo16.astype(jnp.uint32) | (hi16.astype(jnp.uint32) << 16)
    return lax.bitcast_convert_type(word, jnp.float32).reshape(
        _BH, _HD, _WH * _XW
    )


def kernel(x, qkv_w, offset_w, proj_w, proj_b):
    B, C, W, H = x.shape
    heads, hd, ns = _HEADS, _HD, _NS
    N = W * H

    eye = jnp.eye(heads, dtype=jnp.float32)
    blk = eye[:, None, :, None] * offset_w.T[None, :, None, :]
    blk = blk.reshape(C, heads * 2 * ns)
    wcat = jnp.concatenate([qkv_w.T, blk], axis=1)

    xf = jnp.transpose(x, (0, 2, 3, 1)).reshape(B * N, C)
    y = _mm(xf, wcat)

    q = y[:, 0:C].reshape(B, N, heads, hd)
    k = y[:, C:2 * C].reshape(B, N, heads, hd)
    v = y[:, 2 * C:3 * C].reshape(B, N, heads, hd)
    off = y[:, 3 * C:].reshape(B, N, heads, 2 * ns)

    ktab = _pack_pairs(jnp.transpose(k, (0, 2, 3, 1)).reshape(_BH, hd, W, H))
    vtab = _pack_pairs(jnp.transpose(v, (0, 2, 3, 1)).reshape(_BH, hd, W, H))
    q_cm = jnp.transpose(
        q.reshape(B, _NCH, _P, heads, hd), (0, 3, 1, 4, 2)
    ).reshape(_BH, _NCH, hd, _P)
    oy = jnp.transpose(off[..., 0::2], (0, 2, 3, 1)).reshape(_BH, ns, _NCH, _P)
    ox = jnp.transpose(off[..., 1::2], (0, 2, 3, 1)).reshape(_BH, ns, _NCH, _P)
    oy = jnp.transpose(oy, (0, 2, 1, 3))
    ox = jnp.transpose(ox, (0, 2, 1, 3))

    idx_cm, w_cm = _idxw(oy, ox)
    out_cm = _sc_attn(ktab, vtab, q_cm, idx_cm, w_cm)

    out_f = jnp.transpose(
        out_cm.reshape(B, heads, _NCH, hd, _P), (0, 2, 4, 1, 3)
    ).reshape(B * N, C)
    fin = _mm_bias(out_f, proj_w.T, proj_b.reshape(1, C))
    fin = fin.reshape(B, W, H, C)
    return jnp.transpose(fin, (0, 3, 1, 2))


# speedup vs baseline: 2.4956x; 1.6176x over previous
import functools
import jax
import jax.numpy as jnp
from jax import lax
from jax.experimental import pallas as pl
from jax.experimental.pallas import tpu as pltpu
from jax.experimental.pallas import tpu_sc as plsc

_B, _C, _WH = 4, 192, 56
_HEADS, _HD, _NS = 8, 24, 9
_N = _WH * _WH
_NP = 3200
_BH = _B * _HEADS
_P = 128
_NCH = _NP // _P
_SCALE = _HD ** -0.5
_QROW = 2 * _NS * _HEADS
_YR = _QROW + _C


def _colrow(shape, dim):
    i = lax.broadcasted_iota(jnp.int32, shape, dim).astype(jnp.float32)
    r = jnp.floor((i + 0.5) * (1.0 / _WH))
    return i, r, i - _WH * r


def _proj_tables_kernel(x_ref, w_ref, y_ref, kt_ref, vt_ref):
    xb = x_ref[0]
    xp = jnp.pad(xb, ((0, 0), (0, _NP - _N)))
    yb = jnp.dot(w_ref[...], xp, preferred_element_type=jnp.float32)
    y_ref[0] = yb[:_YR]
    _, _, col = _colrow((_C, _N), 1)
    edge = (col == 0.0)
    for rows, out in ((slice(_YR, _YR + _C), kt_ref), (slice(_YR + _C, None), vt_ref)):
        t = yb[rows, :_N]
        lo = jnp.where(edge, 0.0, jnp.pad(t, ((0, 0), (1, 0)))[:, :_N])
        lo16 = lax.bitcast_convert_type(lo.astype(jnp.bfloat16), jnp.uint16)
        hi16 = lax.bitcast_convert_type(t.astype(jnp.bfloat16), jnp.uint16)
        word = lo16.astype(jnp.uint32) | (hi16.astype(jnp.uint32) << 16)
        out[0] = lax.bitcast_convert_type(word, jnp.float32)


def _proj_tables(x3, wcat_t):
    return pl.pallas_call(
        _proj_tables_kernel,
        grid=(_B,),
        in_specs=[
            pl.BlockSpec((1, _C, _N), lambda b: (b, 0, 0)),
            pl.BlockSpec((_YR + 2 * _C, _C), lambda b: (0, 0)),
        ],
        out_specs=[
            pl.BlockSpec((1, _YR, _NP), lambda b: (b, 0, 0)),
            pl.BlockSpec((1, _C, _N), lambda b: (b, 0, 0)),
            pl.BlockSpec((1, _C, _N), lambda b: (b, 0, 0)),
        ],
        out_shape=[
            jax.ShapeDtypeStruct((_B, _YR, _NP), jnp.float32),
            jax.ShapeDtypeStruct((_B, _C, _N), jnp.float32),
            jax.ShapeDtypeStruct((_B, _C, _N), jnp.float32),
        ],
    )(x3, wcat_t)


def _idxw_kernel(off_ref, idx_ref, w_ref):
    shape = (_NS, _NP)
    sf = lax.broadcasted_iota(jnp.int32, shape, 0).astype(jnp.float32)
    _, rowf, colf = _colrow(shape, 1)
    sdiv3 = jnp.floor(sf * (1.0 / 3.0))
    ky = sdiv3 - 1.0
    kx = sf - 3.0 * sdiv3 - 1.0
    for h in range(_HEADS):
        _idxw_one(off_ref[0], h, rowf + ky, colf + kx, idx_ref, w_ref)


def _idxw_one(off, h, base_y, base_x, idx_ref, w_ref):
    o = off[h * 2 * _NS:(h + 1) * 2 * _NS]
    oy = jnp.concatenate([o[2 * s:2 * s + 1] for s in range(_NS)], axis=0)
    ox = jnp.concatenate([o[2 * s + 1:2 * s + 2] for s in range(_NS)], axis=0)
    py = base_y + oy
    px = base_x + ox
    y0 = jnp.floor(py)
    x0 = jnp.floor(px)
    wy1 = py - y0
    wy0 = 1.0 - wy1
    wx1 = px - x0
    wx0 = 1.0 - wx1
    lim = float(_WH - 1)
    y1 = y0 + 1.0
    x1 = x0 + 1.0
    vy0 = ((y0 >= 0) & (y0 <= lim)).astype(jnp.float32)
    vy1 = ((y1 >= 0) & (y1 <= lim)).astype(jnp.float32)
    wx0 = wx0 * ((x0 >= 0) & (x0 <= lim)).astype(jnp.float32)
    wx1 = wx1 * ((x1 >= 0) & (x1 <= lim)).astype(jnp.float32)
    swap = x1 > lim
    xi = jnp.clip(x1, 0.0, lim).astype(jnp.int32)
    wlo = jnp.where(swap, 0.0, wx0)
    whi = jnp.where(swap, wx0, wx1)
    r0 = jnp.clip(y0, 0.0, lim).astype(jnp.int32)
    r1 = jnp.clip(y1, 0.0, lim).astype(jnp.int32)
    idx_ref[0, h] = jnp.concatenate([r0 * _WH + xi, r1 * _WH + xi], axis=0)
    w_ref[0, h] = jnp.concatenate([wy0 * vy0, wy1 * vy1, wlo, whi], axis=0)


def _idxw(y_t):
    idx, w = pl.pallas_call(
        _idxw_kernel,
        grid=(_B,),
        in_specs=[
            pl.BlockSpec((1, _QROW, _NP), lambda b: (b, 0, 0)),
        ],
        out_specs=[
            pl.BlockSpec((1, _HEADS, 2 * _NS, _NP), lambda b: (b, 0, 0, 0)),
            pl.BlockSpec((1, _HEADS, 4 * _NS, _NP), lambda b: (b, 0, 0, 0)),
        ],
        out_shape=[
            jax.ShapeDtypeStruct((_B, _HEADS, 2 * _NS, _NP), jnp.int32),
            jax.ShapeDtypeStruct((_B, _HEADS, 4 * _NS, _NP), jnp.float32),
        ],
    )(y_t)
    return idx.reshape(_BH, 2 * _NS, _NP), w.reshape(_BH, 4 * _NS, _NP)


_SC_MESH = plsc.VectorSubcoreMesh(core_axis_name="c", subcore_axis_name="s")
_SC_PARAMS = pltpu.CompilerParams(needs_layout_passes=False)


def _pair(tab, dsplat, iv):
    g = plsc.load_gather(tab, [dsplat, iv])
    return plsc.unpack(
        plsc.bitcast(g, jnp.bfloat16), format=plsc.PackFormat.INTERLEAVED
    )


@functools.partial(
    pl.kernel,
    out_type=jax.ShapeDtypeStruct((_BH, _HD, _NP), jnp.float32),
    mesh=_SC_MESH,
    compiler_params=_SC_PARAMS,
    scratch_types=[
        pltpu.VMEM((_HD, _N), jnp.float32),
        pltpu.VMEM((_NS - 1, _NP), jnp.float32),
        pltpu.VMEM((_HD + 4 * _NS, _P), jnp.float32),
        pltpu.VMEM((2 * _NS, _P), jnp.int32),
    ],
)
def _sc_attn(ktab_h, vtab_h, y_h, idx_h, w_h, o_h, tab, pball, fb, ib):
    wid = lax.axis_index("s") * 2 + lax.axis_index("c")
    b = wid // _HEADS
    h = wid % _HEADS
    pltpu.sync_copy(ktab_h.at[wid], tab)

    def chunk_a(jc, carry):
        csl = pl.ds(jc * _P, _P)
        pltpu.sync_copy(
            y_h.at[b, pl.ds(_QROW + h * _HD, _HD), csl], fb.at[pl.ds(0, _HD)]
        )
        pltpu.sync_copy(idx_h.at[wid, :, csl], ib)
        pltpu.sync_copy(w_h.at[wid, :, csl], fb.at[pl.ds(_HD, 4 * _NS)])

        def tile(t, carry2):
            sl = pl.ds(t * 16, 16)
            qv = [fb[d, sl] for d in range(_HD)]
            logits = []
            for s in range(_NS):
                iv0 = ib[s, sl]
                iv1 = ib[_NS + s, sl]
                wy0 = fb[_HD + s, sl]
                wy1 = fb[_HD + _NS + s, sl]
                wx0 = fb[_HD + 2 * _NS + s, sl]
                wx1 = fb[_HD + 3 * _NS + s, sl]
                acc = [None] * 4
                for d in range(_HD):
                    dsp = jnp.full((16,), d, jnp.int32)
                    lo0, hi0 = _pair(tab, dsp, iv0)
                    lo1, hi1 = _pair(tab, dsp, iv1)
                    for j, g in enumerate((lo0, hi0, lo1, hi1)):
                        t0 = qv[d] * g
                        acc[j] = t0 if acc[j] is None else acc[j] + t0
                row0 = wx0 * acc[0] + wx1 * acc[1]
                row1 = wx0 * acc[2] + wx1 * acc[3]
                logits.append((wy0 * row0 + wy1 * row1) * _SCALE)
            m = logits[0]
            for s in range(1, _NS):
                m = jnp.maximum(m, logits[s])
            es = [jnp.exp(l - m) for l in logits]
            tot = es[0]
            for s in range(1, _NS):
                tot = tot + es[s]
            gsl = pl.ds(jc * _P + t * 16, 16)
            for s in range(_NS - 1):
                pball[s, gsl] = es[s] / tot
            return carry2

        lax.fori_loop(0, _P // 16, tile, 0)
        return carry

    lax.fori_loop(0, _NCH, chunk_a, 0)
    pltpu.sync_copy(vtab_h.at[wid], tab)

    def chunk_b(jc, carry):
        csl = pl.ds(jc * _P, _P)
        pltpu.sync_copy(idx_h.at[wid, :, csl], ib)
        pltpu.sync_copy(w_h.at[wid, :, csl], fb.at[pl.ds(_HD, 4 * _NS)])

        def tile(t, carry2):
            sl = pl.ds(t * 16, 16)
            gsl = pl.ds(jc * _P + t * 16, 16)
            pv = [pball[s, gsl] for s in range(_NS - 1)]
            plast = 1.0 - pv[0]
            for s in range(1, _NS - 1):
                plast = plast - pv[s]
            pv.append(plast)
            outs = [None] * _HD
            for s in range(_NS):
                iv0 = ib[s, sl]
                iv1 = ib[_NS + s, sl]
                wy0 = fb[_HD + s, sl]
                wy1 = fb[_HD + _NS + s, sl]
                wx0 = fb[_HD + 2 * _NS + s, sl]
                wx1 = fb[_HD + 3 * _NS + s, sl]
                c0 = pv[s] * wy0
                c1 = pv[s] * wy1
                cw = (c0 * wx0, c0 * wx1, c1 * wx0, c1 * wx1)
                for d in range(_HD):
                    dsp = jnp.full((16,), d, jnp.int32)
                    lo0, hi0 = _pair(tab, dsp, iv0)
                    lo1, hi1 = _pair(tab, dsp, iv1)
                    t0 = cw[0] * lo0
                    outs[d] = t0 if outs[d] is None else outs[d] + t0
                    outs[d] = outs[d] + cw[1] * hi0
                    outs[d] = outs[d] + cw[2] * lo1
                    outs[d] = outs[d] + cw[3] * hi1
            for d in range(_HD):
                fb[d, sl] = outs[d]
            return carry2

        lax.fori_loop(0, _P // 16, tile, 0)
        pltpu.sync_copy(fb.at[pl.ds(0, _HD)], o_h.at[wid, :, csl])
        return carry

    lax.fori_loop(0, _NCH, chunk_b, 0)


def _out_proj_kernel(o_ref, w_ref, b_ref, f_ref):
    fin = jnp.dot(w_ref[...], o_ref[0], preferred_element_type=jnp.float32)
    f_ref[0] = fin[:, :_N] + b_ref[...]


def _out_proj(o3, pw, pb):
    return pl.pallas_call(
        _out_proj_kernel,
        grid=(_B,),
        in_specs=[
            pl.BlockSpec((1, _C, _NP), lambda b: (b, 0, 0)),
            pl.BlockSpec((_C, _C), lambda b: (0, 0)),
            pl.BlockSpec((_C, 1), lambda b: (0, 0)),
        ],
        out_specs=pl.BlockSpec((1, _C, _N), lambda b: (b, 0, 0)),
        out_shape=jax.ShapeDtypeStruct((_B, _C, _N), jnp.float32),
    )(o3, pw, pb)


def kernel(x, qkv_w, offset_w, proj_w, proj_b):
    B, C, W, H = x.shape
    heads, hd, ns = _HEADS, _HD, _NS

    eye = jnp.eye(heads, dtype=jnp.float32)
    offt = (eye[:, None, :, None] * offset_w[None, :, None, :]).reshape(
        _QROW, C
    )
    wcat_t = jnp.concatenate([offt, qkv_w], axis=0)

    x3 = x.reshape(B, C, _N)
    y_t, ktab, vtab = _proj_tables(x3, wcat_t)
    idx_t, w_t = _idxw(y_t)
    o_t = _sc_attn(
        ktab.reshape(_BH, hd, _N), vtab.reshape(_BH, hd, _N), y_t, idx_t, w_t
    )
    fin = _out_proj(o_t.reshape(B, C, _NP), proj_w, proj_b.reshape(C, 1))
    return fin.reshape(B, C, W, H)


# speedup vs baseline: 2.6880x; 1.0771x over previous
import functools
import jax
import jax.numpy as jnp
from jax import lax
from jax.experimental import pallas as pl
from jax.experimental.pallas import tpu as pltpu
from jax.experimental.pallas import tpu_sc as plsc

_B, _C, _WH = 4, 192, 56
_HEADS, _HD, _NS = 8, 24, 9
_N = _WH * _WH
_NP = 3200
_BH = _B * _HEADS
_P = 128
_NCH = _NP // _P
_SCALE = _HD ** -0.5
_QROW = 2 * _NS * _HEADS
_YR = _QROW + _C


def _colrow(shape, dim):
    i = lax.broadcasted_iota(jnp.int32, shape, dim).astype(jnp.float32)
    r = jnp.floor((i + 0.5) * (1.0 / _WH))
    return i, r, i - _WH * r


def _proj_tables_kernel(x_ref, w_ref, y_ref, kt_ref, vt_ref):
    xb = x_ref[0]
    xp = jnp.pad(xb, ((0, 0), (0, _NP - _N)))
    yb = jnp.dot(w_ref[...], xp, preferred_element_type=jnp.float32)
    y_ref[0] = yb[:_YR]
    _, _, col = _colrow((_C, _N), 1)
    edge = (col == 0.0)
    for rows, out in ((slice(_YR, _YR + _C), kt_ref), (slice(_YR + _C, None), vt_ref)):
        t = yb[rows, :_N]
        lo = jnp.where(edge, 0.0, jnp.pad(t, ((0, 0), (1, 0)))[:, :_N])
        lo16 = lax.bitcast_convert_type(lo.astype(jnp.bfloat16), jnp.uint16)
        hi16 = lax.bitcast_convert_type(t.astype(jnp.bfloat16), jnp.uint16)
        word = lo16.astype(jnp.uint32) | (hi16.astype(jnp.uint32) << 16)
        out[0] = lax.bitcast_convert_type(word, jnp.float32)


def _proj_tables(x3, wcat_t):
    return pl.pallas_call(
        _proj_tables_kernel,
        grid=(_B,),
        in_specs=[
            pl.BlockSpec((1, _C, _N), lambda b: (b, 0, 0)),
            pl.BlockSpec((_YR + 2 * _C, _C), lambda b: (0, 0)),
        ],
        out_specs=[
            pl.BlockSpec((1, _YR, _NP), lambda b: (b, 0, 0)),
            pl.BlockSpec((1, _C, _N), lambda b: (b, 0, 0)),
            pl.BlockSpec((1, _C, _N), lambda b: (b, 0, 0)),
        ],
        out_shape=[
            jax.ShapeDtypeStruct((_B, _YR, _NP), jnp.float32),
            jax.ShapeDtypeStruct((_B, _C, _N), jnp.float32),
            jax.ShapeDtypeStruct((_B, _C, _N), jnp.float32),
        ],
    )(x3, wcat_t)


def _idxw_kernel(y_ref, qiw_ref):
    shape = (_NS, _NP)
    sf = lax.broadcasted_iota(jnp.int32, shape, 0).astype(jnp.float32)
    _, rowf, colf = _colrow(shape, 1)
    sdiv3 = jnp.floor(sf * (1.0 / 3.0))
    ky = sdiv3 - 1.0
    kx = sf - 3.0 * sdiv3 - 1.0
    for h in range(_HEADS):
        qiw_ref[0, h, 0:_HD] = y_ref[0, _QROW + h * _HD:_QROW + (h + 1) * _HD]
        _idxw_one(y_ref[0], h, rowf + ky, colf + kx, qiw_ref)


def _idxw_one(off, h, base_y, base_x, qiw_ref):
    o = off[h * 2 * _NS:(h + 1) * 2 * _NS]
    oy = jnp.concatenate([o[2 * s:2 * s + 1] for s in range(_NS)], axis=0)
    ox = jnp.concatenate([o[2 * s + 1:2 * s + 2] for s in range(_NS)], axis=0)
    py = base_y + oy
    px = base_x + ox
    y0 = jnp.floor(py)
    x0 = jnp.floor(px)
    wy1 = py - y0
    wy0 = 1.0 - wy1
    wx1 = px - x0
    wx0 = 1.0 - wx1
    lim = float(_WH - 1)
    y1 = y0 + 1.0
    x1 = x0 + 1.0
    vy0 = ((y0 >= 0) & (y0 <= lim)).astype(jnp.float32)
    vy1 = ((y1 >= 0) & (y1 <= lim)).astype(jnp.float32)
    wx0 = wx0 * ((x0 >= 0) & (x0 <= lim)).astype(jnp.float32)
    wx1 = wx1 * ((x1 >= 0) & (x1 <= lim)).astype(jnp.float32)
    swap = x1 > lim
    xi = jnp.clip(x1, 0.0, lim).astype(jnp.int32)
    wlo = jnp.where(swap, 0.0, wx0)
    whi = jnp.where(swap, wx0, wx1)
    r0 = jnp.clip(y0, 0.0, lim).astype(jnp.int32)
    r1 = jnp.clip(y1, 0.0, lim).astype(jnp.int32)
    idx = jnp.concatenate([r0 * _WH + xi, r1 * _WH + xi], axis=0)
    qiw_ref[0, h, _HD:_HD + 2 * _NS] = lax.bitcast_convert_type(
        idx, jnp.float32
    )
    qiw_ref[0, h, _HD + 2 * _NS:] = jnp.concatenate(
        [wy0 * vy0, wy1 * vy1, wlo, whi], axis=0
    )


_QIWR = _HD + 6 * _NS


def _idxw(y_t):
    qiw = pl.pallas_call(
        _idxw_kernel,
        grid=(_B,),
        in_specs=[
            pl.BlockSpec((1, _YR, _NP), lambda b: (b, 0, 0)),
        ],
        out_specs=pl.BlockSpec((1, _HEADS, _QIWR, _NP), lambda b: (b, 0, 0, 0)),
        out_shape=jax.ShapeDtypeStruct((_B, _HEADS, _QIWR, _NP), jnp.float32),
    )(y_t)
    return qiw.reshape(_BH, _QIWR, _NP)


_SC_MESH = plsc.VectorSubcoreMesh(core_axis_name="c", subcore_axis_name="s")
_SC_PARAMS = pltpu.CompilerParams(needs_layout_passes=False)


def _pair(tab, dsplat, iv):
    g = plsc.load_gather(tab, [dsplat, iv])
    return plsc.unpack(
        plsc.bitcast(g, jnp.bfloat16), format=plsc.PackFormat.INTERLEAVED
    )


_WROW = _HD + 2 * _NS


@functools.partial(
    pl.kernel,
    out_type=jax.ShapeDtypeStruct((_BH, _HD, _NP), jnp.float32),
    mesh=_SC_MESH,
    compiler_params=_SC_PARAMS,
    scratch_types=[
        pltpu.VMEM((_HD, _N), jnp.float32),
        pltpu.VMEM((_NS - 1, _NP), jnp.float32),
        pltpu.VMEM((_QIWR, _P), jnp.float32),
    ],
)
def _sc_attn(ktab_h, vtab_h, qiw_h, o_h, tab, pball, fb):
    wid = lax.axis_index("s") * 2 + lax.axis_index("c")
    pltpu.sync_copy(ktab_h.at[wid], tab)

    def chunk_a(jc, carry):
        csl = pl.ds(jc * _P, _P)
        pltpu.sync_copy(qiw_h.at[wid, :, csl], fb)

        def tile(t, carry2):
            sl = pl.ds(t * 16, 16)
            qv = [fb[d, sl] for d in range(_HD)]
            logits = []
            for s in range(_NS):
                iv0 = plsc.bitcast(fb[_HD + s, sl], jnp.int32)
                iv1 = plsc.bitcast(fb[_HD + _NS + s, sl], jnp.int32)
                wy0 = fb[_WROW + s, sl]
                wy1 = fb[_WROW + _NS + s, sl]
                wx0 = fb[_WROW + 2 * _NS + s, sl]
                wx1 = fb[_WROW + 3 * _NS + s, sl]
                acc = [None] * 4
                for d in range(_HD):
                    dsp = jnp.full((16,), d, jnp.int32)
                    lo0, hi0 = _pair(tab, dsp, iv0)
                    lo1, hi1 = _pair(tab, dsp, iv1)
                    for j, g in enumerate((lo0, hi0, lo1, hi1)):
                        t0 = qv[d] * g
                        acc[j] = t0 if acc[j] is None else acc[j] + t0
                row0 = wx0 * acc[0] + wx1 * acc[1]
                row1 = wx0 * acc[2] + wx1 * acc[3]
                logits.append((wy0 * row0 + wy1 * row1) * _SCALE)
            m = logits[0]
            for s in range(1, _NS):
                m = jnp.maximum(m, logits[s])
            es = [jnp.exp(l - m) for l in logits]
            tot = es[0]
            for s in range(1, _NS):
                tot = tot + es[s]
            gsl = pl.ds(jc * _P + t * 16, 16)
            for s in range(_NS - 1):
                pball[s, gsl] = es[s] / tot
            return carry2

        lax.fori_loop(0, _P // 16, tile, 0)
        return carry

    lax.fori_loop(0, _NCH, chunk_a, 0)
    pltpu.sync_copy(vtab_h.at[wid], tab)

    def chunk_b(jc, carry):
        csl = pl.ds(jc * _P, _P)
        pltpu.sync_copy(qiw_h.at[wid, :, csl], fb)

        def tile(t, carry2):
            sl = pl.ds(t * 16, 16)
            gsl = pl.ds(jc * _P + t * 16, 16)
            pv = [pball[s, gsl] for s in range(_NS - 1)]
            plast = 1.0 - pv[0]
            for s in range(1, _NS - 1):
                plast = plast - pv[s]
            pv.append(plast)
            outs = [None] * _HD
            for s in range(_NS):
                iv0 = plsc.bitcast(fb[_HD + s, sl], jnp.int32)
                iv1 = plsc.bitcast(fb[_HD + _NS + s, sl], jnp.int32)
                wy0 = fb[_WROW + s, sl]
                wy1 = fb[_WROW + _NS + s, sl]
                wx0 = fb[_WROW + 2 * _NS + s, sl]
                wx1 = fb[_WROW + 3 * _NS + s, sl]
                c0 = pv[s] * wy0
                c1 = pv[s] * wy1
                cw = (c0 * wx0, c0 * wx1, c1 * wx0, c1 * wx1)
                for d in range(_HD):
                    dsp = jnp.full((16,), d, jnp.int32)
                    lo0, hi0 = _pair(tab, dsp, iv0)
                    lo1, hi1 = _pair(tab, dsp, iv1)
                    t0 = cw[0] * lo0
                    outs[d] = t0 if outs[d] is None else outs[d] + t0
                    outs[d] = outs[d] + cw[1] * hi0
                    outs[d] = outs[d] + cw[2] * lo1
                    outs[d] = outs[d] + cw[3] * hi1
            for d in range(_HD):
                fb[d, sl] = outs[d]
            return carry2

        lax.fori_loop(0, _P // 16, tile, 0)
        pltpu.sync_copy(fb.at[pl.ds(0, _HD)], o_h.at[wid, :, csl])
        return carry

    lax.fori_loop(0, _NCH, chunk_b, 0)


def _out_proj_kernel(o_ref, w_ref, b_ref, f_ref):
    fin = jnp.dot(w_ref[...], o_ref[0], preferred_element_type=jnp.float32)
    f_ref[0] = fin[:, :_N] + b_ref[...]


def _out_proj(o3, pw, pb):
    return pl.pallas_call(
        _out_proj_kernel,
        grid=(_B,),
        in_specs=[
            pl.BlockSpec((1, _C, _NP), lambda b: (b, 0, 0)),
            pl.BlockSpec((_C, _C), lambda b: (0, 0)),
            pl.BlockSpec((_C, 1), lambda b: (0, 0)),
        ],
        out_specs=pl.BlockSpec((1, _C, _N), lambda b: (b, 0, 0)),
        out_shape=jax.ShapeDtypeStruct((_B, _C, _N), jnp.float32),
    )(o3, pw, pb)


def kernel(x, qkv_w, offset_w, proj_w, proj_b):
    B, C, W, H = x.shape
    heads, hd, ns = _HEADS, _HD, _NS

    eye = jnp.eye(heads, dtype=jnp.float32)
    offt = (eye[:, None, :, None] * offset_w[None, :, None, :]).reshape(
        _QROW, C
    )
    wcat_t = jnp.concatenate([offt, qkv_w], axis=0)

    x3 = x.reshape(B, C, _N)
    y_t, ktab, vtab = _proj_tables(x3, wcat_t)
    qiw = _idxw(y_t)
    o_t = _sc_attn(ktab.reshape(_BH, hd, _N), vtab.reshape(_BH, hd, _N), qiw)
    fin = _out_proj(o_t.reshape(B, C, _NP), proj_w, proj_b.reshape(C, 1))
    return fin.reshape(B, C, W, H)


# speedup vs baseline: 2.8832x; 1.0726x over previous
import functools
import jax
import jax.numpy as jnp
from jax import lax
from jax.experimental import pallas as pl
from jax.experimental.pallas import tpu as pltpu
from jax.experimental.pallas import tpu_sc as plsc

_B, _C, _WH = 4, 192, 56
_HEADS, _HD, _NS = 8, 24, 9
_N = _WH * _WH
_NP = 3200
_BH = _B * _HEADS
_P = 128
_NCH = _NP // _P
_SCALE = _HD ** -0.5
_QROW = 2 * _NS * _HEADS
_YR = _QROW + _C


def _colrow(shape, dim):
    i = lax.broadcasted_iota(jnp.int32, shape, dim).astype(jnp.float32)
    r = jnp.floor((i + 0.5) * (1.0 / _WH))
    return i, r, i - _WH * r


def _proj_tables_kernel(x_ref, w_ref, y_ref, kt_ref, vt_ref):
    xb = x_ref[0]
    xp = jnp.pad(xb, ((0, 0), (0, _NP - _N)))
    yb = jnp.dot(w_ref[...], xp, preferred_element_type=jnp.float32)
    y_ref[0] = yb[:_YR]
    _, _, col = _colrow((_C, _N), 1)
    edge = (col == 0.0)
    for rows, out in ((slice(_YR, _YR + _C), kt_ref), (slice(_YR + _C, None), vt_ref)):
        t = yb[rows, :_N]
        lo = jnp.where(edge, 0.0, jnp.pad(t, ((0, 0), (1, 0)))[:, :_N])
        lo16 = lax.bitcast_convert_type(lo.astype(jnp.bfloat16), jnp.uint16)
        hi16 = lax.bitcast_convert_type(t.astype(jnp.bfloat16), jnp.uint16)
        word = lo16.astype(jnp.uint32) | (hi16.astype(jnp.uint32) << 16)
        out[0] = lax.bitcast_convert_type(word, jnp.float32)


def _proj_tables(x3, wcat_t):
    return pl.pallas_call(
        _proj_tables_kernel,
        grid=(_B,),
        in_specs=[
            pl.BlockSpec((1, _C, _N), lambda b: (b, 0, 0)),
            pl.BlockSpec((_YR + 2 * _C, _C), lambda b: (0, 0)),
        ],
        out_specs=[
            pl.BlockSpec((1, _YR, _NP), lambda b: (b, 0, 0)),
            pl.BlockSpec((1, _C, _N), lambda b: (b, 0, 0)),
            pl.BlockSpec((1, _C, _N), lambda b: (b, 0, 0)),
        ],
        out_shape=[
            jax.ShapeDtypeStruct((_B, _YR, _NP), jnp.float32),
            jax.ShapeDtypeStruct((_B, _C, _N), jnp.float32),
            jax.ShapeDtypeStruct((_B, _C, _N), jnp.float32),
        ],
    )(x3, wcat_t)


def _idxw_kernel(y_ref, qiw_ref):
    shape = (_NS, _NP)
    sf = lax.broadcasted_iota(jnp.int32, shape, 0).astype(jnp.float32)
    _, rowf, colf = _colrow(shape, 1)
    sdiv3 = jnp.floor(sf * (1.0 / 3.0))
    ky = sdiv3 - 1.0
    kx = sf - 3.0 * sdiv3 - 1.0
    for h in range(_HEADS):
        qiw_ref[0, h, 0:_HD] = y_ref[0, _QROW + h * _HD:_QROW + (h + 1) * _HD]
        _idxw_one(y_ref[0], h, rowf + ky, colf + kx, qiw_ref)


def _idxw_one(off, h, base_y, base_x, qiw_ref):
    o = off[h * 2 * _NS:(h + 1) * 2 * _NS]
    oy = jnp.concatenate([o[2 * s:2 * s + 1] for s in range(_NS)], axis=0)
    ox = jnp.concatenate([o[2 * s + 1:2 * s + 2] for s in range(_NS)], axis=0)
    py = base_y + oy
    px = base_x + ox
    y0 = jnp.floor(py)
    x0 = jnp.floor(px)
    wy1 = py - y0
    wy0 = 1.0 - wy1
    wx1 = px - x0
    wx0 = 1.0 - wx1
    lim = float(_WH - 1)
    y1 = y0 + 1.0
    x1 = x0 + 1.0
    vy0 = ((y0 >= 0) & (y0 <= lim)).astype(jnp.float32)
    vy1 = ((y1 >= 0) & (y1 <= lim)).astype(jnp.float32)
    wx0 = wx0 * ((x0 >= 0) & (x0 <= lim)).astype(jnp.float32)
    wx1 = wx1 * ((x1 >= 0) & (x1 <= lim)).astype(jnp.float32)
    swap = x1 > lim
    xi = jnp.clip(x1, 0.0, lim).astype(jnp.int32)
    wlo = jnp.where(swap, 0.0, wx0)
    whi = jnp.where(swap, wx0, wx1)
    r0 = jnp.clip(y0, 0.0, lim).astype(jnp.int32)
    r1 = jnp.clip(y1, 0.0, lim).astype(jnp.int32)
    idx = jnp.concatenate([r0 * _WH + xi, r1 * _WH + xi], axis=0)
    qiw_ref[0, h, _HD:_HD + 2 * _NS] = lax.bitcast_convert_type(
        idx, jnp.float32
    )
    qiw_ref[0, h, _HD + 2 * _NS:] = jnp.concatenate(
        [wy0 * vy0, wy1 * vy1, wlo, whi], axis=0
    )


_QIWR = _HD + 6 * _NS


def _idxw(y_t):
    qiw = pl.pallas_call(
        _idxw_kernel,
        grid=(_B,),
        in_specs=[
            pl.BlockSpec((1, _YR, _NP), lambda b: (b, 0, 0)),
        ],
        out_specs=pl.BlockSpec((1, _HEADS, _QIWR, _NP), lambda b: (b, 0, 0, 0)),
        out_shape=jax.ShapeDtypeStruct((_B, _HEADS, _QIWR, _NP), jnp.float32),
    )(y_t)
    return qiw.reshape(_BH, _QIWR, _NP)


_SC_MESH = plsc.VectorSubcoreMesh(core_axis_name="c", subcore_axis_name="s")
_SC_PARAMS = pltpu.CompilerParams(needs_layout_passes=False)


def _pair(tab, dsplat, iv):
    g = plsc.load_gather(tab, [dsplat, iv])
    return plsc.unpack(
        plsc.bitcast(g, jnp.bfloat16), format=plsc.PackFormat.INTERLEAVED
    )


_WROW = _HD + 2 * _NS
_NPAIR = _NCH // 2


def _compute_a(tab, fb, pb):
    def tile(t, carry2):
        sl = pl.ds(t * 16, 16)
        qv = [fb[d, sl] for d in range(_HD)]
        logits = []
        for s in range(_NS):
            iv0 = plsc.bitcast(fb[_HD + s, sl], jnp.int32)
            iv1 = plsc.bitcast(fb[_HD + _NS + s, sl], jnp.int32)
            wy0 = fb[_WROW + s, sl]
            wy1 = fb[_WROW + _NS + s, sl]
            wx0 = fb[_WROW + 2 * _NS + s, sl]
            wx1 = fb[_WROW + 3 * _NS + s, sl]
            acc = [None] * 4
            for d in range(_HD):
                dsp = jnp.full((16,), d, jnp.int32)
                lo0, hi0 = _pair(tab, dsp, iv0)
                lo1, hi1 = _pair(tab, dsp, iv1)
                for j, g in enumerate((lo0, hi0, lo1, hi1)):
                    t0 = qv[d] * g
                    acc[j] = t0 if acc[j] is None else acc[j] + t0
            row0 = wx0 * acc[0] + wx1 * acc[1]
            row1 = wx0 * acc[2] + wx1 * acc[3]
            logits.append((wy0 * row0 + wy1 * row1) * _SCALE)
        m = logits[0]
        for s in range(1, _NS):
            m = jnp.maximum(m, logits[s])
        es = [jnp.exp(l - m) for l in logits]
        tot = es[0]
        for s in range(1, _NS):
            tot = tot + es[s]
        for s in range(_NS - 1):
            pb[s, sl] = es[s] / tot
        return carry2

    lax.fori_loop(0, _P // 16, tile, 0)


def _compute_b(tab, fb, pb, ob):
    def tile(t, carry2):
        sl = pl.ds(t * 16, 16)
        pv = [pb[s, sl] for s in range(_NS - 1)]
        plast = 1.0 - pv[0]
        for s in range(1, _NS - 1):
            plast = plast - pv[s]
        pv.append(plast)
        outs = [None] * _HD
        for s in range(_NS):
            iv0 = plsc.bitcast(fb[_HD + s, sl], jnp.int32)
            iv1 = plsc.bitcast(fb[_HD + _NS + s, sl], jnp.int32)
            wy0 = fb[_WROW + s, sl]
            wy1 = fb[_WROW + _NS + s, sl]
            wx0 = fb[_WROW + 2 * _NS + s, sl]
            wx1 = fb[_WROW + 3 * _NS + s, sl]
            c0 = pv[s] * wy0
            c1 = pv[s] * wy1
            cw = (c0 * wx0, c0 * wx1, c1 * wx0, c1 * wx1)
            for d in range(_HD):
                dsp = jnp.full((16,), d, jnp.int32)
                lo0, hi0 = _pair(tab, dsp, iv0)
                lo1, hi1 = _pair(tab, dsp, iv1)
                t0 = cw[0] * lo0
                outs[d] = t0 if outs[d] is None else outs[d] + t0
                outs[d] = outs[d] + cw[1] * hi0
                outs[d] = outs[d] + cw[2] * lo1
                outs[d] = outs[d] + cw[3] * hi1
        for d in range(_HD):
            ob[d, sl] = outs[d]
        return carry2

    lax.fori_loop(0, _P // 16, tile, 0)


@functools.partial(
    pl.kernel,
    out_type=(
        jax.ShapeDtypeStruct((_BH, _HD, _NP), jnp.float32),
        jax.ShapeDtypeStruct((_BH, _NS - 1, _NP), jnp.float32),
    ),
    mesh=_SC_MESH,
    compiler_params=_SC_PARAMS,
    scratch_types=[
        pltpu.VMEM((_HD, _N), jnp.float32),
        pltpu.VMEM((_QIWR, _P), jnp.float32),
        pltpu.VMEM((_QIWR, _P), jnp.float32),
        pltpu.VMEM((_NS - 1, _P), jnp.float32),
        pltpu.VMEM((_NS - 1, _P), jnp.float32),
        pltpu.VMEM((_HD, _P), jnp.float32),
        pltpu.VMEM((_HD, _P), jnp.float32),
        pltpu.SemaphoreType.DMA,
        pltpu.SemaphoreType.DMA,
        pltpu.SemaphoreType.DMA,
        pltpu.SemaphoreType.DMA,
        pltpu.SemaphoreType.DMA,
        pltpu.SemaphoreType.DMA,
    ],
)
def _sc_attn(
    ktab_h, vtab_h, qiw_h, o_h, p_h,
    tab, fb0, fb1, pb0, pb1, ob0, ob1, cin0, cin1, pio0, pio1, ow0, ow1
):
    wid = lax.axis_index("s") * 2 + lax.axis_index("c")

    def csl(jc):
        return pl.ds(jc * _P, _P)

    def in_start(jc, fb, sem):
        pltpu.async_copy(qiw_h.at[wid, :, csl(jc)], fb, sem)

    def in_wait(fb, sem):
        pltpu.make_async_copy(qiw_h.at[wid, :, csl(0)], fb, sem).wait()

    def p_write(jc, pb, sem):
        pltpu.async_copy(pb, p_h.at[wid, :, csl(jc)], sem)

    def p_read(jc, pb, sem):
        pltpu.async_copy(p_h.at[wid, :, csl(jc)], pb, sem)

    def p_wait(pb, sem):
        pltpu.make_async_copy(pb, p_h.at[wid, :, csl(0)], sem).wait()

    def o_write(jc, ob, sem):
        pltpu.async_copy(ob, o_h.at[wid, :, csl(jc)], sem)

    def o_wait(ob, sem):
        pltpu.make_async_copy(ob, o_h.at[wid, :, csl(0)], sem).wait()

    pltpu.sync_copy(ktab_h.at[wid], tab)
    in_start(0, fb0, cin0)
    in_start(1, fb1, cin1)

    def pair_a(i, carry):
        jc0 = 2 * i
        for fb, pb, cin, pio, jc, more in (
            (fb0, pb0, cin0, pio0, jc0, True),
            (fb1, pb1, cin1, pio1, jc0 + 1, False),
        ):
            in_wait(fb, cin)

            @pl.when(i > 0)
            def _():
                p_wait(pb, pio)

            _compute_a(tab, fb, pb)
            p_write(jc, pb, pio)
            if more:
                in_start(jc + 2, fb, cin)
            else:
                @pl.when(i < _NPAIR - 1)
                def _():
                    in_start(jc + 2, fb, cin)
        return carry

    lax.fori_loop(0, _NPAIR, pair_a, 0)
    in_wait(fb0, cin0)
    p_wait(pb0, pio0)
    _compute_a(tab, fb0, pb0)
    p_write(_NCH - 1, pb0, pio0)
    p_wait(pb0, pio0)
    p_wait(pb1, pio1)

    pltpu.sync_copy(vtab_h.at[wid], tab)
    in_start(0, fb0, cin0)
    in_start(1, fb1, cin1)
    p_read(0, pb0, pio0)
    p_read(1, pb1, pio1)

    def pair_b(i, carry):
        jc0 = 2 * i
        for fb, pb, ob, cin, pio, ow, jc, more in (
            (fb0, pb0, ob0, cin0, pio0, ow0, jc0, True),
            (fb1, pb1, ob1, cin1, pio1, ow1, jc0 + 1, False),
        ):
            in_wait(fb, cin)
            p_wait(pb, pio)

            @pl.when(i > 0)
            def _():
                o_wait(ob, ow)

            _compute_b(tab, fb, pb, ob)
            o_write(jc, ob, ow)
            if more:
                in_start(jc + 2, fb, cin)
                p_read(jc + 2, pb, pio)
            else:
                @pl.when(i < _NPAIR - 1)
                def _():
                    in_start(jc + 2, fb, cin)
                    p_read(jc + 2, pb, pio)
        return carry

    lax.fori_loop(0, _NPAIR, pair_b, 0)
    in_wait(fb0, cin0)
    p_wait(pb0, pio0)
    o_wait(ob0, ow0)
    _compute_b(tab, fb0, pb0, ob0)
    o_write(_NCH - 1, ob0, ow0)
    o_wait(ob0, ow0)
    o_wait(ob1, ow1)


def _out_proj_kernel(o_ref, w_ref, b_ref, f_ref):
    fin = jnp.dot(w_ref[...], o_ref[0], preferred_element_type=jnp.float32)
    f_ref[0] = fin[:, :_N] + b_ref[...]


def _out_proj(o3, pw, pb):
    return pl.pallas_call(
        _out_proj_kernel,
        grid=(_B,),
        in_specs=[
            pl.BlockSpec((1, _C, _NP), lambda b: (b, 0, 0)),
            pl.BlockSpec((_C, _C), lambda b: (0, 0)),
            pl.BlockSpec((_C, 1), lambda b: (0, 0)),
        ],
        out_specs=pl.BlockSpec((1, _C, _N), lambda b: (b, 0, 0)),
        out_shape=jax.ShapeDtypeStruct((_B, _C, _N), jnp.float32),
    )(o3, pw, pb)


def kernel(x, qkv_w, offset_w, proj_w, proj_b):
    B, C, W, H = x.shape
    heads, hd, ns = _HEADS, _HD, _NS

    eye = jnp.eye(heads, dtype=jnp.float32)
    offt = (eye[:, None, :, None] * offset_w[None, :, None, :]).reshape(
        _QROW, C
    )
    wcat_t = jnp.concatenate([offt, qkv_w], axis=0)

    x3 = x.reshape(B, C, _N)
    y_t, ktab, vtab = _proj_tables(x3, wcat_t)
    qiw = _idxw(y_t)
    o_t, _ = _sc_attn(ktab.reshape(_BH, hd, _N), vtab.reshape(_BH, hd, _N), qiw)
    fin = _out_proj(o_t.reshape(B, C, _NP), proj_w, proj_b.reshape(C, 1))
    return fin.reshape(B, C, W, H)


# speedup vs baseline: 2.8870x; 1.0013x over previous
import functools
import jax
import jax.numpy as jnp
from jax import lax
from jax.experimental import pallas as pl
from jax.experimental.pallas import tpu as pltpu
from jax.experimental.pallas import tpu_sc as plsc

_B, _C, _WH = 4, 192, 56
_HEADS, _HD, _NS = 8, 24, 9
_N = _WH * _WH
_NP = 3200
_BH = _B * _HEADS
_P = 128
_NCH = _NP // _P
_SCALE = _HD ** -0.5
_QROW = 2 * _NS * _HEADS
_YR = _QROW + _C


def _colrow(shape, dim):
    i = lax.broadcasted_iota(jnp.int32, shape, dim).astype(jnp.float32)
    r = jnp.floor((i + 0.5) * (1.0 / _WH))
    return i, r, i - _WH * r


def _proj_tables_kernel(x_ref, w_ref, y_ref, kt_ref, vt_ref):
    xb = x_ref[0]
    xp = jnp.pad(xb, ((0, 0), (0, _NP - _N)))
    yb = jnp.dot(w_ref[...], xp, preferred_element_type=jnp.float32)
    y_ref[0] = yb[:_YR]
    _, _, col = _colrow((_C, _N), 1)
    edge = (col == 0.0)
    for rows, out in ((slice(_YR, _YR + _C), kt_ref), (slice(_YR + _C, None), vt_ref)):
        t = yb[rows, :_N]
        lo = jnp.where(edge, 0.0, jnp.pad(t, ((0, 0), (1, 0)))[:, :_N])
        lo16 = lax.bitcast_convert_type(lo.astype(jnp.bfloat16), jnp.uint16)
        hi16 = lax.bitcast_convert_type(t.astype(jnp.bfloat16), jnp.uint16)
        word = lo16.astype(jnp.uint32) | (hi16.astype(jnp.uint32) << 16)
        out[0] = lax.bitcast_convert_type(word, jnp.float32)


def _proj_tables(x3, wcat_t):
    return pl.pallas_call(
        _proj_tables_kernel,
        grid=(_B,),
        in_specs=[
            pl.BlockSpec((1, _C, _N), lambda b: (b, 0, 0)),
            pl.BlockSpec((_YR + 2 * _C, _C), lambda b: (0, 0)),
        ],
        out_specs=[
            pl.BlockSpec((1, _YR, _NP), lambda b: (b, 0, 0)),
            pl.BlockSpec((1, _C, _N), lambda b: (b, 0, 0)),
            pl.BlockSpec((1, _C, _N), lambda b: (b, 0, 0)),
        ],
        out_shape=[
            jax.ShapeDtypeStruct((_B, _YR, _NP), jnp.float32),
            jax.ShapeDtypeStruct((_B, _C, _N), jnp.float32),
            jax.ShapeDtypeStruct((_B, _C, _N), jnp.float32),
        ],
    )(x3, wcat_t)


def _idxw_kernel(y_ref, qiw_ref):
    shape = (_NS, _NP)
    sf = lax.broadcasted_iota(jnp.int32, shape, 0).astype(jnp.float32)
    _, rowf, colf = _colrow(shape, 1)
    sdiv3 = jnp.floor(sf * (1.0 / 3.0))
    ky = sdiv3 - 1.0
    kx = sf - 3.0 * sdiv3 - 1.0
    for h in range(_HEADS):
        qiw_ref[0, h, 0:_HD] = y_ref[0, _QROW + h * _HD:_QROW + (h + 1) * _HD]
        _idxw_one(y_ref[0], h, rowf + ky, colf + kx, qiw_ref)


def _idxw_one(off, h, base_y, base_x, qiw_ref):
    o = off[h * 2 * _NS:(h + 1) * 2 * _NS]
    oy = jnp.concatenate([o[2 * s:2 * s + 1] for s in range(_NS)], axis=0)
    ox = jnp.concatenate([o[2 * s + 1:2 * s + 2] for s in range(_NS)], axis=0)
    py = base_y + oy
    px = base_x + ox
    y0 = jnp.floor(py)
    x0 = jnp.floor(px)
    wy1 = py - y0
    wy0 = 1.0 - wy1
    wx1 = px - x0
    wx0 = 1.0 - wx1
    lim = float(_WH - 1)
    y1 = y0 + 1.0
    x1 = x0 + 1.0
    vy0 = ((y0 >= 0) & (y0 <= lim)).astype(jnp.float32)
    vy1 = ((y1 >= 0) & (y1 <= lim)).astype(jnp.float32)
    wx0 = wx0 * ((x0 >= 0) & (x0 <= lim)).astype(jnp.float32)
    wx1 = wx1 * ((x1 >= 0) & (x1 <= lim)).astype(jnp.float32)
    swap = x1 > lim
    xi = jnp.clip(x1, 0.0, lim).astype(jnp.int32)
    wlo = jnp.where(swap, 0.0, wx0)
    whi = jnp.where(swap, wx0, wx1)
    r0 = jnp.clip(y0, 0.0, lim).astype(jnp.int32)
    r1 = jnp.clip(y1, 0.0, lim).astype(jnp.int32)
    idx = jnp.concatenate([r0 * _WH + xi, r1 * _WH + xi], axis=0)
    qiw_ref[0, h, _HD:_HD + 2 * _NS] = lax.bitcast_convert_type(
        idx, jnp.float32
    )
    qiw_ref[0, h, _HD + 2 * _NS:] = jnp.concatenate(
        [wy0 * vy0, wy1 * vy1, wlo, whi], axis=0
    )


_QIWR = _HD + 6 * _NS


def _idxw(y_t):
    qiw = pl.pallas_call(
        _idxw_kernel,
        grid=(_B,),
        in_specs=[
            pl.BlockSpec((1, _YR, _NP), lambda b: (b, 0, 0)),
        ],
        out_specs=pl.BlockSpec((1, _HEADS, _QIWR, _NP), lambda b: (b, 0, 0, 0)),
        out_shape=jax.ShapeDtypeStruct((_B, _HEADS, _QIWR, _NP), jnp.float32),
    )(y_t)
    return qiw.reshape(_BH, _QIWR, _NP)


_SC_MESH = plsc.VectorSubcoreMesh(core_axis_name="c", subcore_axis_name="s")
_SC_PARAMS = pltpu.CompilerParams(needs_layout_passes=False)


def _pair(tab, dsplat, iv):
    g = plsc.load_gather(tab, [dsplat, iv])
    return plsc.unpack(
        plsc.bitcast(g, jnp.bfloat16), format=plsc.PackFormat.INTERLEAVED
    )


_WROW = _HD + 2 * _NS
_NPAIR = _NCH // 2


def _compute_a(tab, fb, pb):
    def tile_a(t):
        sl = pl.ds(t * 16, 16)
        qv = [fb[d, sl] for d in range(_HD)]
        logits = []
        for s in range(_NS):
            iv0 = plsc.bitcast(fb[_HD + s, sl], jnp.int32)
            iv1 = plsc.bitcast(fb[_HD + _NS + s, sl], jnp.int32)
            wy0 = fb[_WROW + s, sl]
            wy1 = fb[_WROW + _NS + s, sl]
            wx0 = fb[_WROW + 2 * _NS + s, sl]
            wx1 = fb[_WROW + 3 * _NS + s, sl]
            acc = [None] * 4
            for d in range(_HD):
                dsp = jnp.full((16,), d, jnp.int32)
                lo0, hi0 = _pair(tab, dsp, iv0)
                lo1, hi1 = _pair(tab, dsp, iv1)
                for j, g in enumerate((lo0, hi0, lo1, hi1)):
                    t0 = qv[d] * g
                    acc[j] = t0 if acc[j] is None else acc[j] + t0
            row0 = wx0 * acc[0] + wx1 * acc[1]
            row1 = wx0 * acc[2] + wx1 * acc[3]
            logits.append((wy0 * row0 + wy1 * row1) * _SCALE)
        m = logits[0]
        for s in range(1, _NS):
            m = jnp.maximum(m, logits[s])
        es = [jnp.exp(l - m) for l in logits]
        tot = es[0]
        for s in range(1, _NS):
            tot = tot + es[s]
        for s in range(_NS - 1):
            pb[s, sl] = es[s] / tot

    plsc.parallel_loop(0, _P // 16)(tile_a)


def _compute_b(tab, fb, pb, ob):
    def tile_b(t):
        sl = pl.ds(t * 16, 16)
        pv = [pb[s, sl] for s in range(_NS - 1)]
        plast = 1.0 - pv[0]
        for s in range(1, _NS - 1):
            plast = plast - pv[s]
        pv.append(plast)
        outs = [None] * _HD
        for s in range(_NS):
            iv0 = plsc.bitcast(fb[_HD + s, sl], jnp.int32)
            iv1 = plsc.bitcast(fb[_HD + _NS + s, sl], jnp.int32)
            wy0 = fb[_WROW + s, sl]
            wy1 = fb[_WROW + _NS + s, sl]
            wx0 = fb[_WROW + 2 * _NS + s, sl]
            wx1 = fb[_WROW + 3 * _NS + s, sl]
            c0 = pv[s] * wy0
            c1 = pv[s] * wy1
            cw = (c0 * wx0, c0 * wx1, c1 * wx0, c1 * wx1)
            for d in range(_HD):
                dsp = jnp.full((16,), d, jnp.int32)
                lo0, hi0 = _pair(tab, dsp, iv0)
                lo1, hi1 = _pair(tab, dsp, iv1)
                t0 = cw[0] * lo0
                outs[d] = t0 if outs[d] is None else outs[d] + t0
                outs[d] = outs[d] + cw[1] * hi0
                outs[d] = outs[d] + cw[2] * lo1
                outs[d] = outs[d] + cw[3] * hi1
        for d in range(_HD):
            ob[d, sl] = outs[d]

    plsc.parallel_loop(0, _P // 16)(tile_b)


@functools.partial(
    pl.kernel,
    out_type=(
        jax.ShapeDtypeStruct((_BH, _HD, _NP), jnp.float32),
        jax.ShapeDtypeStruct((_BH, _NS - 1, _NP), jnp.float32),
    ),
    mesh=_SC_MESH,
    compiler_params=_SC_PARAMS,
    scratch_types=[
        pltpu.VMEM((_HD, _N), jnp.float32),
        pltpu.VMEM((_QIWR, _P), jnp.float32),
        pltpu.VMEM((_QIWR, _P), jnp.float32),
        pltpu.VMEM((_NS - 1, _P), jnp.float32),
        pltpu.VMEM((_NS - 1, _P), jnp.float32),
        pltpu.VMEM((_HD, _P), jnp.float32),
        pltpu.VMEM((_HD, _P), jnp.float32),
        pltpu.SemaphoreType.DMA,
        pltpu.SemaphoreType.DMA,
        pltpu.SemaphoreType.DMA,
        pltpu.SemaphoreType.DMA,
        pltpu.SemaphoreType.DMA,
        pltpu.SemaphoreType.DMA,
    ],
)
def _sc_attn(
    ktab_h, vtab_h, qiw_h, o_h, p_h,
    tab, fb0, fb1, pb0, pb1, ob0, ob1, cin0, cin1, pio0, pio1, ow0, ow1
):
    wid = lax.axis_index("s") * 2 + lax.axis_index("c")

    def csl(jc):
        return pl.ds(jc * _P, _P)

    def in_start(jc, fb, sem):
        pltpu.async_copy(qiw_h.at[wid, :, csl(jc)], fb, sem)

    def in_wait(fb, sem):
        pltpu.make_async_copy(qiw_h.at[wid, :, csl(0)], fb, sem).wait()

    def p_write(jc, pb, sem):
        pltpu.async_copy(pb, p_h.at[wid, :, csl(jc)], sem)

    def p_read(jc, pb, sem):
        pltpu.async_copy(p_h.at[wid, :, csl(jc)], pb, sem)

    def p_wait(pb, sem):
        pltpu.make_async_copy(pb, p_h.at[wid, :, csl(0)], sem).wait()

    def o_write(jc, ob, sem):
        pltpu.async_copy(ob, o_h.at[wid, :, csl(jc)], sem)

    def o_wait(ob, sem):
        pltpu.make_async_copy(ob, o_h.at[wid, :, csl(0)], sem).wait()

    pltpu.sync_copy(ktab_h.at[wid], tab)
    in_start(0, fb0, cin0)
    in_start(1, fb1, cin1)

    def pair_a(i, carry):
        jc0 = 2 * i
        for fb, pb, cin, pio, jc, more in (
            (fb0, pb0, cin0, pio0, jc0, True),
            (fb1, pb1, cin1, pio1, jc0 + 1, False),
        ):
            in_wait(fb, cin)

            @pl.when(i > 0)
            def _():
                p_wait(pb, pio)

            _compute_a(tab, fb, pb)
            p_write(jc, pb, pio)
            if more:
                in_start(jc + 2, fb, cin)
            else:
                @pl.when(i < _NPAIR - 1)
                def _():
                    in_start(jc + 2, fb, cin)
        return carry

    lax.fori_loop(0, _NPAIR, pair_a, 0)
    in_wait(fb0, cin0)
    p_wait(pb0, pio0)
    _compute_a(tab, fb0, pb0)
    p_write(_NCH - 1, pb0, pio0)
    p_wait(pb0, pio0)
    p_wait(pb1, pio1)

    pltpu.sync_copy(vtab_h.at[wid], tab)
    in_start(0, fb0, cin0)
    in_start(1, fb1, cin1)
    p_read(0, pb0, pio0)
    p_read(1, pb1, pio1)

    def pair_b(i, carry):
        jc0 = 2 * i
        for fb, pb, ob, cin, pio, ow, jc, more in (
            (fb0, pb0, ob0, cin0, pio0, ow0, jc0, True),
            (fb1, pb1, ob1, cin1, pio1, ow1, jc0 + 1, False),
        ):
            in_wait(fb, cin)
            p_wait(pb, pio)

            @pl.when(i > 0)
            def _():
                o_wait(ob, ow)

            _compute_b(tab, fb, pb, ob)
            o_write(jc, ob, ow)
            if more:
                in_start(jc + 2, fb, cin)
                p_read(jc + 2, pb, pio)
            else:
                @pl.when(i < _NPAIR - 1)
                def _():
                    in_start(jc + 2, fb, cin)
                    p_read(jc + 2, pb, pio)
        return carry

    lax.fori_loop(0, _NPAIR, pair_b, 0)
    in_wait(fb0, cin0)
    p_wait(pb0, pio0)
    o_wait(ob0, ow0)
    _compute_b(tab, fb0, pb0, ob0)
    o_write(_NCH - 1, ob0, ow0)
    o_wait(ob0, ow0)
    o_wait(ob1, ow1)


def _out_proj_kernel(o_ref, w_ref, b_ref, f_ref):
    fin = jnp.dot(w_ref[...], o_ref[0], preferred_element_type=jnp.float32)
    f_ref[0] = fin[:, :_N] + b_ref[...]


def _out_proj(o3, pw, pb):
    return pl.pallas_call(
        _out_proj_kernel,
        grid=(_B,),
        in_specs=[
            pl.BlockSpec((1, _C, _NP), lambda b: (b, 0, 0)),
            pl.BlockSpec((_C, _C), lambda b: (0, 0)),
            pl.BlockSpec((_C, 1), lambda b: (0, 0)),
        ],
        out_specs=pl.BlockSpec((1, _C, _N), lambda b: (b, 0, 0)),
        out_shape=jax.ShapeDtypeStruct((_B, _C, _N), jnp.float32),
    )(o3, pw, pb)


def kernel(x, qkv_w, offset_w, proj_w, proj_b):
    B, C, W, H = x.shape
    heads, hd, ns = _HEADS, _HD, _NS

    eye = jnp.eye(heads, dtype=jnp.float32)
    offt = (eye[:, None, :, None] * offset_w[None, :, None, :]).reshape(
        _QROW, C
    )
    wcat_t = jnp.concatenate([offt, qkv_w], axis=0)

    x3 = x.reshape(B, C, _N)
    y_t, ktab, vtab = _proj_tables(x3, wcat_t)
    qiw = _idxw(y_t)
    o_t, _ = _sc_attn(ktab.reshape(_BH, hd, _N), vtab.reshape(_BH, hd, _N), qiw)
    fin = _out_proj(o_t.reshape(B, C, _NP), proj_w, proj_b.reshape(C, 1))
    return fin.reshape(B, C, W, H)
